# baseline (speedup 1.0000x reference)
import functools

import jax
import jax.numpy as jnp
from jax import lax
from jax.experimental import pallas as pl
from jax.experimental.pallas import tpu as pltpu
from jax.experimental.pallas import tpu_sc as plsc

N_NODES = 10000
N_EDGES = 320000
IN_DIM = 128
HIDDEN = 256
OUT_DIM = 128
NUM_CAT, NUM_INGR, NUM_SHOP = 64, 1024, 256
EMB = 16
FUSE = 24
D1 = IN_DIM + FUSE

NC = 2
NS = 16
NW = NC * NS
CHUNK = 128
E_PER_TILE = N_EDGES // NW
N_PAIRS = 39
TAIL = 16
DEG_W = 128
ZROWS = N_NODES // NS

T1_BLK = 1000
T1_STEPS = N_NODES // T1_BLK


@functools.cache
def _make_sc_deg():
    mesh = plsc.VectorSubcoreMesh(core_axis_name="c", subcore_axis_name="s")

    @functools.partial(
        pl.kernel, mesh=mesh,
        out_type=jax.ShapeDtypeStruct((NC, N_NODES, DEG_W), jnp.float32),
        scratch_types=[
            pltpu.VMEM((CHUNK,), jnp.int32),
            pltpu.VMEM((CHUNK,), jnp.int32),
            pltpu.VMEM((TAIL,), jnp.int32),
            pltpu.VMEM((CHUNK, DEG_W), jnp.float32),
            pltpu.VMEM_SHARED((N_NODES, DEG_W), jnp.float32),
            pltpu.SemaphoreType.DMA,
            pltpu.SemaphoreType.DMA,
        ],
    )
    def sc_deg(dst_h, deg_out, di0, di1, dit, fill_v, acc, sem0, sem1):
        cid = lax.axis_index("c")
        sid = lax.axis_index("s")
        wid = cid * NS + sid
        base = wid * E_PER_TILE

        zeros16 = jnp.zeros((16,), jnp.float32)
        ones16 = jnp.ones((16,), jnp.float32)

        def fill(vec16):
            def st(i, c):
                fill_v[i // (DEG_W // 16),
                       pl.ds((i % (DEG_W // 16)) * 16, 16)] = vec16
                return c
            lax.fori_loop(0, CHUNK * (DEG_W // 16), st, 0)

        fill(zeros16)

        @pl.when(sid < 15)
        def _():
            def z(j, c):
                pltpu.sync_copy(fill_v,
                                acc.at[pl.ds(sid * 624 + j * CHUNK, CHUNK)])
                return c
            lax.fori_loop(0, 4, z, 0)
            pltpu.sync_copy(fill_v.at[pl.ds(0, 112)],
                            acc.at[pl.ds(sid * 624 + 512, 112)])

        @pl.when(sid == 15)
        def _():
            def z(j, c):
                pltpu.sync_copy(fill_v,
                                acc.at[pl.ds(9360 + j * CHUNK, CHUNK)])
                return c
            lax.fori_loop(0, 5, z, 0)

        fill(ones16)
        plsc.subcore_barrier()

        def deg_body(g, c):
            b0 = base + (2 * g) * CHUNK
            h0 = pltpu.async_copy(dst_h.at[pl.ds(b0, CHUNK)], di0, sem0)
            h1 = pltpu.async_copy(dst_h.at[pl.ds(b0 + CHUNK, CHUNK)], di1,
                                  sem1)
            h0.wait()
            pltpu.sync_copy(fill_v, acc.at[di0], add=True)
            h1.wait()
            pltpu.sync_copy(fill_v, acc.at[di1], add=True)
            return c
        lax.fori_loop(0, N_PAIRS, deg_body, 0)

        pltpu.sync_copy(dst_h.at[pl.ds(base + 2 * N_PAIRS * CHUNK, TAIL)], dit)
        pltpu.sync_copy(fill_v.at[pl.ds(0, TAIL)], acc.at[dit], add=True)
        plsc.subcore_barrier()

        @pl.when(sid < 15)
        def _():
            pltpu.sync_copy(acc.at[pl.ds(sid * 624, 624)],
                            deg_out.at[cid, pl.ds(sid * 624, 624)])

        @pl.when(sid == 15)
        def _():
            pltpu.sync_copy(acc.at[pl.ds(9360, 640)],
                            deg_out.at[cid, pl.ds(9360, 640)])

    return sc_deg


D = 128


@functools.cache
def _make_sc_agg():
    mesh = plsc.VectorSubcoreMesh(core_axis_name="c", subcore_axis_name="s")

    @functools.partial(
        pl.kernel, mesh=mesh,
        out_type=jax.ShapeDtypeStruct((NC, N_NODES, D), jnp.float32),
        scratch_types=[
            pltpu.VMEM((CHUNK,), jnp.int32),
            pltpu.VMEM((CHUNK,), jnp.int32),
            pltpu.VMEM((CHUNK,), jnp.int32),
            pltpu.VMEM((CHUNK,), jnp.int32),
            pltpu.VMEM((TAIL,), jnp.int32),
            pltpu.VMEM((TAIL,), jnp.int32),
            pltpu.VMEM((CHUNK, D), jnp.float32),
            pltpu.VMEM((CHUNK, D), jnp.float32),
            pltpu.VMEM_SHARED((N_NODES, D), jnp.float32),
            pltpu.SemaphoreType.DMA,
            pltpu.SemaphoreType.DMA,
            pltpu.SemaphoreType.DMA,
            pltpu.SemaphoreType.DMA,
            pltpu.SemaphoreType.DMA,
            pltpu.SemaphoreType.DMA,
        ],
    )
    def agg(val_h, src_h, dst_h, out_h, si0, si1, di0, di1, sit, dit,
            r0, r1, acc, ss0, ss1, sd0, sd1, sg0, sg1):
        cid = lax.axis_index("c")
        sid = lax.axis_index("s")
        wid = cid * NS + sid
        base = wid * E_PER_TILE

        @pl.when(sid < 15)
        def _():
            pltpu.sync_copy(val_h.at[pl.ds(sid * 624, 624)],
                            acc.at[pl.ds(sid * 624, 624)])

        @pl.when(sid == 15)
        def _():
            pltpu.sync_copy(val_h.at[pl.ds(9360, 640)],
                            acc.at[pl.ds(9360, 640)])
        plsc.subcore_barrier()

        def body(g, c):
            b0 = base + (2 * g) * CHUNK
            hs0 = pltpu.async_copy(src_h.at[pl.ds(b0, CHUNK)], si0, ss0)
            hd0 = pltpu.async_copy(dst_h.at[pl.ds(b0, CHUNK)], di0, sd0)
            hs1 = pltpu.async_copy(src_h.at[pl.ds(b0 + CHUNK, CHUNK)], si1,
                                   ss1)
            hd1 = pltpu.async_copy(dst_h.at[pl.ds(b0 + CHUNK, CHUNK)], di1,
                                   sd1)
            hs0.wait()
            hg0 = pltpu.async_copy(val_h.at[si0], r0, sg0)
            hs1.wait()
            hg1 = pltpu.async_copy(val_h.at[si1], r1, sg1)
            hg0.wait()
            hd0.wait()
            pltpu.sync_copy(r0, acc.at[di0], add=True)
            hg1.wait()
            hd1.wait()
            pltpu.sync_copy(r1, acc.at[di1], add=True)
            return c
        lax.fori_loop(0, N_PAIRS, body, 0)

        bt = base + 2 * N_PAIRS * CHUNK
        pltpu.sync_copy(src_h.at[pl.ds(bt, TAIL)], sit)
        pltpu.async_copy(val_h.at[sit], r0.at[pl.ds(0, TAIL)], sg0).wait()
        pltpu.sync_copy(dst_h.at[pl.ds(bt, TAIL)], dit)
        pltpu.sync_copy(r0.at[pl.ds(0, TAIL)], acc.at[dit], add=True)
        plsc.subcore_barrier()

        @pl.when(sid < 15)
        def _():
            pltpu.sync_copy(acc.at[pl.ds(sid * 624, 624)],
                            out_h.at[cid, pl.ds(sid * 624, 624)])

        @pl.when(sid == 15)
        def _():
            pltpu.sync_copy(acc.at[pl.ds(9360, 640)],
                            out_h.at[cid, pl.ds(9360, 640)])

    return agg


E2_PER_TILE = N_EDGES // NS
N2_PAIRS = 78
TAIL2 = 32


@functools.cache
def _make_sc_agg_cols():
    mesh = plsc.VectorSubcoreMesh(core_axis_name="c", subcore_axis_name="s")

    @functools.partial(
        pl.kernel, mesh=mesh,
        out_type=jax.ShapeDtypeStruct((NC, N_NODES, D), jnp.float32),
        scratch_types=[
            pltpu.VMEM((CHUNK,), jnp.int32),
            pltpu.VMEM((CHUNK,), jnp.int32),
            pltpu.VMEM((CHUNK,), jnp.int32),
            pltpu.VMEM((CHUNK,), jnp.int32),
            pltpu.VMEM((TAIL2,), jnp.int32),
            pltpu.VMEM((TAIL2,), jnp.int32),
            pltpu.VMEM((CHUNK, D), jnp.float32),
            pltpu.VMEM((CHUNK, D), jnp.float32),
            pltpu.VMEM_SHARED((N_NODES, D), jnp.float32),
            pltpu.SemaphoreType.DMA,
            pltpu.SemaphoreType.DMA,
            pltpu.SemaphoreType.DMA,
            pltpu.SemaphoreType.DMA,
            pltpu.SemaphoreType.DMA,
            pltpu.SemaphoreType.DMA,
        ],
    )
    def agg2(vals_h, src_h, dst_h, out_h, si0, si1, di0, di1, sit, dit,
             r0, r1, acc, ss0, ss1, sd0, sd1, sg0, sg1):
        cid = lax.axis_index("c")
        sid = lax.axis_index("s")
        base = sid * E2_PER_TILE
        val_h = vals_h.at[cid]

        @pl.when(sid < 15)
        def _():
            pltpu.sync_copy(val_h.at[pl.ds(sid * 624, 624)],
                            acc.at[pl.ds(sid * 624, 624)])

        @pl.when(sid == 15)
        def _():
            pltpu.sync_copy(val_h.at[pl.ds(9360, 640)],
                            acc.at[pl.ds(9360, 640)])
        plsc.subcore_barrier()

        def body(g, c):
            b0 = base + (2 * g) * CHUNK
            hs0 = pltpu.async_copy(src_h.at[pl.ds(b0, CHUNK)], si0, ss0)
            hd0 = pltpu.async_copy(dst_h.at[pl.ds(b0, CHUNK)], di0, sd0)
            hs1 = pltpu.async_copy(src_h.at[pl.ds(b0 + CHUNK, CHUNK)], si1,
                                   ss1)
            hd1 = pltpu.async_copy(dst_h.at[pl.ds(b0 + CHUNK, CHUNK)], di1,
                                   sd1)
            hs0.wait()
            hg0 = pltpu.async_copy(val_h.at[si0], r0, sg0)
            hs1.wait()
            hg1 = pltpu.async_copy(val_h.at[si1], r1, sg1)
            hg0.wait()
            hd0.wait()
            pltpu.sync_copy(r0, acc.at[di0], add=True)
            hg1.wait()
            hd1.wait()
            pltpu.sync_copy(r1, acc.at[di1], add=True)
            return c
        lax.fori_loop(0, N2_PAIRS, body, 0)

        bt = base + 2 * N2_PAIRS * CHUNK
        pltpu.sync_copy(src_h.at[pl.ds(bt, TAIL2)], sit)
        pltpu.async_copy(val_h.at[sit], r0.at[pl.ds(0, TAIL2)], sg0).wait()
        pltpu.sync_copy(dst_h.at[pl.ds(bt, TAIL2)], dit)
        pltpu.sync_copy(r0.at[pl.ds(0, TAIL2)], acc.at[dit], add=True)
        plsc.subcore_barrier()

        @pl.when(sid < 15)
        def _():
            pltpu.sync_copy(acc.at[pl.ds(sid * 624, 624)],
                            out_h.at[cid, pl.ds(sid * 624, 624)])

        @pl.when(sid == 15)
        def _():
            pltpu.sync_copy(acc.at[pl.ds(9360, 640)],
                            out_h.at[cid, pl.ds(9360, 640)])

    return agg2


def _t1_body(x_ref, ids_ref, d_ref, ce_ref, ie_ref, se_ref, wf_ref, bf_ref,
             w1_ref, out_ref):
    deg = jnp.sum(d_ref[...], axis=1) + 1.0
    dinv = lax.rsqrt(deg)

    ids = ids_ref[...]
    oh_c = (lax.broadcasted_iota(jnp.int32, (T1_BLK, NUM_CAT), 1)
            == ids[:, 0:1]).astype(jnp.float32)
    oh_i = (lax.broadcasted_iota(jnp.int32, (T1_BLK, NUM_INGR), 1)
            == ids[:, 1:2]).astype(jnp.float32)
    oh_s = (lax.broadcasted_iota(jnp.int32, (T1_BLK, NUM_SHOP), 1)
            == ids[:, 2:3]).astype(jnp.float32)
    e_c = jnp.dot(oh_c, ce_ref[...], preferred_element_type=jnp.float32)
    e_i = jnp.dot(oh_i, ie_ref[...], preferred_element_type=jnp.float32)
    e_s = jnp.dot(oh_s, se_ref[...], preferred_element_type=jnp.float32)

    fused_in = jnp.concatenate([e_c, e_i, e_s], axis=1)
    fused = jnp.dot(fused_in, wf_ref[...],
                    preferred_element_type=jnp.float32,
                    precision=lax.Precision.HIGHEST) + bf_ref[...]
    fused = jnp.maximum(fused, 0.0)
    h0 = jnp.concatenate([x_ref[...], fused], axis=1)
    val1 = jnp.dot(h0, w1_ref[...],
                   preferred_element_type=jnp.float32,
                   precision=lax.Precision.HIGHEST) * dinv[:, None]
    out_ref[0, :, :] = val1[:, :D]
    out_ref[1, :, :] = val1[:, D:]


def _t2_body(g_ref, d_ref, b1_ref, bn_g_ref, bn_b_ref, w2_ref, out_ref):
    deg = jnp.sum(d_ref[...], axis=1) + 1.0
    dinv = lax.rsqrt(deg)

    h1 = g_ref[0] * dinv[:, None] + b1_ref[0]
    mean = jnp.mean(h1, axis=0)
    var = jnp.mean((h1 - mean) ** 2, axis=0)
    hbn = (h1 - mean) * lax.rsqrt(var + 1e-5) * bn_g_ref[0] + bn_b_ref[0]
    hbn = jnp.maximum(hbn, 0.0)
    xw2 = jnp.dot(hbn, w2_ref[...],
                  preferred_element_type=jnp.float32,
                  precision=lax.Precision.HIGHEST) * dinv[:, None]

    @pl.when(pl.program_id(0) == 0)
    def _():
        out_ref[...] = xw2

    @pl.when(pl.program_id(0) != 0)
    def _():
        out_ref[...] += xw2


def _t3_body(acc_ref, val2_ref, d_ref, b2_ref, out_ref):
    deg = jnp.sum(d_ref[...], axis=1) + 1.0
    dinv = lax.rsqrt(deg)
    a = acc_ref[...]
    out_ref[...] = (a[0] + a[1] - val2_ref[...]) * dinv[:, None] + b2_ref[...]


def kernel(x, edge_index, cat_id, ingr_id, shop_id, cat_emb, ingr_emb,
           shop_emb, Wf, bf, W1, b1, gamma, beta, W2, b2):
    src = edge_index[0].astype(jnp.int32)
    dst = edge_index[1].astype(jnp.int32)
    ids = jnp.stack([cat_id.astype(jnp.int32), ingr_id.astype(jnp.int32),
                     shop_id.astype(jnp.int32)], axis=1)

    dacc = _make_sc_deg()(dst)
    deg2 = dacc[:, :, 0].T

    vals1 = pl.pallas_call(
        _t1_body,
        grid=(T1_STEPS,),
        in_specs=[
            pl.BlockSpec((T1_BLK, IN_DIM), lambda i: (i, 0)),
            pl.BlockSpec((T1_BLK, 3), lambda i: (i, 0)),
            pl.BlockSpec((T1_BLK, NC), lambda i: (i, 0)),
            pl.BlockSpec((NUM_CAT, EMB), lambda i: (0, 0)),
            pl.BlockSpec((NUM_INGR, EMB), lambda i: (0, 0)),
            pl.BlockSpec((NUM_SHOP, EMB), lambda i: (0, 0)),
            pl.BlockSpec((EMB * 3, FUSE), lambda i: (0, 0)),
            pl.BlockSpec((1, FUSE), lambda i: (0, 0)),
            pl.BlockSpec((D1, HIDDEN), lambda i: (0, 0)),
        ],
        out_specs=pl.BlockSpec((NC, T1_BLK, D), lambda i: (0, i, 0)),
        out_shape=jax.ShapeDtypeStruct((NC, N_NODES, D), jnp.float32),
    )(x, ids, deg2, cat_emb, ingr_emb, shop_emb, Wf, bf.reshape(1, FUSE), W1)

    gs = _make_sc_agg_cols()(vals1, src, dst)
    val2 = pl.pallas_call(
        _t2_body,
        grid=(2,),
        in_specs=[
            pl.BlockSpec((1, N_NODES, D), lambda j: (j, 0, 0)),
            pl.BlockSpec((N_NODES, NC), lambda j: (0, 0)),
            pl.BlockSpec((1, 1, D), lambda j: (j, 0, 0)),
            pl.BlockSpec((1, 1, D), lambda j: (j, 0, 0)),
            pl.BlockSpec((1, 1, D), lambda j: (j, 0, 0)),
            pl.BlockSpec((D, OUT_DIM), lambda j: (j, 0)),
        ],
        out_specs=pl.BlockSpec((N_NODES, OUT_DIM), lambda j: (0, 0)),
        out_shape=jax.ShapeDtypeStruct((N_NODES, OUT_DIM), jnp.float32),
    )(gs, deg2, b1.reshape(2, 1, D), gamma.reshape(2, 1, D),
      beta.reshape(2, 1, D), W2)

    acc2 = _make_sc_agg()(val2, src, dst)

    out = pl.pallas_call(
        _t3_body,
        out_shape=jax.ShapeDtypeStruct((N_NODES, OUT_DIM), jnp.float32),
    )(acc2, val2, deg2, b2.reshape(1, OUT_DIM))
    return out


# speedup vs baseline: 1.0427x; 1.0427x over previous
import functools

import jax
import jax.numpy as jnp
from jax import lax
from jax.experimental import pallas as pl
from jax.experimental.pallas import tpu as pltpu
from jax.experimental.pallas import tpu_sc as plsc

N_NODES = 10000
N_EDGES = 320000
IN_DIM = 128
HIDDEN = 256
OUT_DIM = 128
NUM_CAT, NUM_INGR, NUM_SHOP = 64, 1024, 256
EMB = 16
FUSE = 24
D1 = IN_DIM + FUSE

NC = 2
NS = 16
NW = NC * NS
CHUNK = 128
E_PER_TILE = N_EDGES // NW
N_PAIRS = 39
TAIL = 16
DEG_W = 128
ZROWS = N_NODES // NS

T1_BLK = 1000
T1_STEPS = N_NODES // T1_BLK


@functools.cache
def _make_sc_deg():
    mesh = plsc.VectorSubcoreMesh(core_axis_name="c", subcore_axis_name="s")

    @functools.partial(
        pl.kernel, mesh=mesh,
        out_type=jax.ShapeDtypeStruct((NC, N_NODES, DEG_W), jnp.float32),
        scratch_types=[
            pltpu.VMEM((CHUNK,), jnp.int32),
            pltpu.VMEM((CHUNK,), jnp.int32),
            pltpu.VMEM((TAIL,), jnp.int32),
            pltpu.VMEM((CHUNK, DEG_W), jnp.float32),
            pltpu.VMEM_SHARED((N_NODES, DEG_W), jnp.float32),
            pltpu.SemaphoreType.DMA,
            pltpu.SemaphoreType.DMA,
        ],
    )
    def sc_deg(dst_h, deg_out, di0, di1, dit, fill_v, acc, sem0, sem1):
        cid = lax.axis_index("c")
        sid = lax.axis_index("s")
        wid = cid * NS + sid
        base = wid * E_PER_TILE

        zeros16 = jnp.zeros((16,), jnp.float32)
        ones16 = jnp.ones((16,), jnp.float32)

        def fill(vec16):
            def st(i, c):
                fill_v[i // (DEG_W // 16),
                       pl.ds((i % (DEG_W // 16)) * 16, 16)] = vec16
                return c
            lax.fori_loop(0, CHUNK * (DEG_W // 16), st, 0)

        fill(zeros16)

        @pl.when(sid < 15)
        def _():
            def z(j, c):
                pltpu.sync_copy(fill_v,
                                acc.at[pl.ds(sid * 624 + j * CHUNK, CHUNK)])
                return c
            lax.fori_loop(0, 4, z, 0)
            pltpu.sync_copy(fill_v.at[pl.ds(0, 112)],
                            acc.at[pl.ds(sid * 624 + 512, 112)])

        @pl.when(sid == 15)
        def _():
            def z(j, c):
                pltpu.sync_copy(fill_v,
                                acc.at[pl.ds(9360 + j * CHUNK, CHUNK)])
                return c
            lax.fori_loop(0, 5, z, 0)

        fill(ones16)
        plsc.subcore_barrier()

        def deg_body(g, c):
            b0 = base + (2 * g) * CHUNK
            h0 = pltpu.async_copy(dst_h.at[pl.ds(b0, CHUNK)], di0, sem0)
            h1 = pltpu.async_copy(dst_h.at[pl.ds(b0 + CHUNK, CHUNK)], di1,
                                  sem1)
            h0.wait()
            pltpu.sync_copy(fill_v, acc.at[di0], add=True)
            h1.wait()
            pltpu.sync_copy(fill_v, acc.at[di1], add=True)
            return c
        lax.fori_loop(0, N_PAIRS, deg_body, 0)

        pltpu.sync_copy(dst_h.at[pl.ds(base + 2 * N_PAIRS * CHUNK, TAIL)], dit)
        pltpu.sync_copy(fill_v.at[pl.ds(0, TAIL)], acc.at[dit], add=True)
        plsc.subcore_barrier()

        @pl.when(sid < 15)
        def _():
            pltpu.sync_copy(acc.at[pl.ds(sid * 624, 624)],
                            deg_out.at[cid, pl.ds(sid * 624, 624)])

        @pl.when(sid == 15)
        def _():
            pltpu.sync_copy(acc.at[pl.ds(9360, 640)],
                            deg_out.at[cid, pl.ds(9360, 640)])

    return sc_deg


D = 128


@functools.cache
def _make_sc_agg():
    mesh = plsc.VectorSubcoreMesh(core_axis_name="c", subcore_axis_name="s")

    @functools.partial(
        pl.kernel, mesh=mesh,
        out_type=jax.ShapeDtypeStruct((NC, N_NODES, D), jnp.float32),
        scratch_types=[
            pltpu.VMEM((CHUNK,), jnp.int32),
            pltpu.VMEM((CHUNK,), jnp.int32),
            pltpu.VMEM((CHUNK,), jnp.int32),
            pltpu.VMEM((CHUNK,), jnp.int32),
            pltpu.VMEM((TAIL,), jnp.int32),
            pltpu.VMEM((TAIL,), jnp.int32),
            pltpu.VMEM((CHUNK, D), jnp.float32),
            pltpu.VMEM((CHUNK, D), jnp.float32),
            pltpu.VMEM_SHARED((N_NODES, D), jnp.float32),
            pltpu.SemaphoreType.DMA,
            pltpu.SemaphoreType.DMA,
            pltpu.SemaphoreType.DMA,
            pltpu.SemaphoreType.DMA,
            pltpu.SemaphoreType.DMA,
            pltpu.SemaphoreType.DMA,
        ],
    )
    def agg(val_h, src_h, dst_h, out_h, si0, si1, di0, di1, sit, dit,
            r0, r1, acc, ss0, ss1, sd0, sd1, sg0, sg1):
        cid = lax.axis_index("c")
        sid = lax.axis_index("s")
        wid = cid * NS + sid
        base = wid * E_PER_TILE

        @pl.when(sid < 15)
        def _():
            pltpu.sync_copy(val_h.at[pl.ds(sid * 624, 624)],
                            acc.at[pl.ds(sid * 624, 624)])

        @pl.when(sid == 15)
        def _():
            pltpu.sync_copy(val_h.at[pl.ds(9360, 640)],
                            acc.at[pl.ds(9360, 640)])
        plsc.subcore_barrier()

        def body(g, c):
            b0 = base + (2 * g) * CHUNK
            hs0 = pltpu.async_copy(src_h.at[pl.ds(b0, CHUNK)], si0, ss0)
            hd0 = pltpu.async_copy(dst_h.at[pl.ds(b0, CHUNK)], di0, sd0)
            hs1 = pltpu.async_copy(src_h.at[pl.ds(b0 + CHUNK, CHUNK)], si1,
                                   ss1)
            hd1 = pltpu.async_copy(dst_h.at[pl.ds(b0 + CHUNK, CHUNK)], di1,
                                   sd1)
            hs0.wait()
            hg0 = pltpu.async_copy(val_h.at[si0], r0, sg0)
            hs1.wait()
            hg1 = pltpu.async_copy(val_h.at[si1], r1, sg1)
            hg0.wait()
            hd0.wait()
            pltpu.sync_copy(r0, acc.at[di0], add=True)
            hg1.wait()
            hd1.wait()
            pltpu.sync_copy(r1, acc.at[di1], add=True)
            return c
        lax.fori_loop(0, N_PAIRS, body, 0)

        bt = base + 2 * N_PAIRS * CHUNK
        pltpu.sync_copy(src_h.at[pl.ds(bt, TAIL)], sit)
        pltpu.async_copy(val_h.at[sit], r0.at[pl.ds(0, TAIL)], sg0).wait()
        pltpu.sync_copy(dst_h.at[pl.ds(bt, TAIL)], dit)
        pltpu.sync_copy(r0.at[pl.ds(0, TAIL)], acc.at[dit], add=True)
        plsc.subcore_barrier()

        @pl.when(sid < 15)
        def _():
            pltpu.sync_copy(acc.at[pl.ds(sid * 624, 624)],
                            out_h.at[cid, pl.ds(sid * 624, 624)])

        @pl.when(sid == 15)
        def _():
            pltpu.sync_copy(acc.at[pl.ds(9360, 640)],
                            out_h.at[cid, pl.ds(9360, 640)])

    return agg


E2_PER_TILE = N_EDGES // NS
N2_PAIRS = 78
TAIL2 = 32


@functools.cache
def _make_sc_agg_cols():
    mesh = plsc.VectorSubcoreMesh(core_axis_name="c", subcore_axis_name="s")

    @functools.partial(
        pl.kernel, mesh=mesh,
        out_type=jax.ShapeDtypeStruct((NC, N_NODES, D), jnp.float32),
        scratch_types=[
            pltpu.VMEM((CHUNK,), jnp.int32),
            pltpu.VMEM((CHUNK,), jnp.int32),
            pltpu.VMEM((CHUNK,), jnp.int32),
            pltpu.VMEM((CHUNK,), jnp.int32),
            pltpu.VMEM((TAIL2,), jnp.int32),
            pltpu.VMEM((TAIL2,), jnp.int32),
            pltpu.VMEM((CHUNK, D), jnp.float32),
            pltpu.VMEM((CHUNK, D), jnp.float32),
            pltpu.VMEM_SHARED((N_NODES, D), jnp.float32),
            pltpu.SemaphoreType.DMA,
            pltpu.SemaphoreType.DMA,
            pltpu.SemaphoreType.DMA,
            pltpu.SemaphoreType.DMA,
            pltpu.SemaphoreType.DMA,
            pltpu.SemaphoreType.DMA,
        ],
    )
    def agg2(vals_h, src_h, dst_h, out_h, si0, si1, di0, di1, sit, dit,
             r0, r1, acc, ss0, ss1, sd0, sd1, sg0, sg1):
        cid = lax.axis_index("c")
        sid = lax.axis_index("s")
        base = sid * E2_PER_TILE
        val_h = vals_h.at[cid]

        @pl.when(sid < 15)
        def _():
            pltpu.sync_copy(val_h.at[pl.ds(sid * 624, 624)],
                            acc.at[pl.ds(sid * 624, 624)])

        @pl.when(sid == 15)
        def _():
            pltpu.sync_copy(val_h.at[pl.ds(9360, 640)],
                            acc.at[pl.ds(9360, 640)])
        plsc.subcore_barrier()

        def body(g, c):
            b0 = base + (2 * g) * CHUNK
            hs0 = pltpu.async_copy(src_h.at[pl.ds(b0, CHUNK)], si0, ss0)
            hd0 = pltpu.async_copy(dst_h.at[pl.ds(b0, CHUNK)], di0, sd0)
            hs1 = pltpu.async_copy(src_h.at[pl.ds(b0 + CHUNK, CHUNK)], si1,
                                   ss1)
            hd1 = pltpu.async_copy(dst_h.at[pl.ds(b0 + CHUNK, CHUNK)], di1,
                                   sd1)
            hs0.wait()
            hg0 = pltpu.async_copy(val_h.at[si0], r0, sg0)
            hs1.wait()
            hg1 = pltpu.async_copy(val_h.at[si1], r1, sg1)
            hg0.wait()
            hd0.wait()
            pltpu.sync_copy(r0, acc.at[di0], add=True)
            hg1.wait()
            hd1.wait()
            pltpu.sync_copy(r1, acc.at[di1], add=True)
            return c
        lax.fori_loop(0, N2_PAIRS, body, 0)

        bt = base + 2 * N2_PAIRS * CHUNK
        pltpu.sync_copy(src_h.at[pl.ds(bt, TAIL2)], sit)
        pltpu.async_copy(val_h.at[sit], r0.at[pl.ds(0, TAIL2)], sg0).wait()
        pltpu.sync_copy(dst_h.at[pl.ds(bt, TAIL2)], dit)
        pltpu.sync_copy(r0.at[pl.ds(0, TAIL2)], acc.at[dit], add=True)
        plsc.subcore_barrier()

        @pl.when(sid < 15)
        def _():
            pltpu.sync_copy(acc.at[pl.ds(sid * 624, 624)],
                            out_h.at[cid, pl.ds(sid * 624, 624)])

        @pl.when(sid == 15)
        def _():
            pltpu.sync_copy(acc.at[pl.ds(9360, 640)],
                            out_h.at[cid, pl.ds(9360, 640)])

    return agg2


def _t1_body(x_ref, ids_ref, ce_ref, ie_ref, se_ref, wf_ref, bf_ref,
             w1_ref, out_ref):
    ids = ids_ref[...]
    oh_c = (lax.broadcasted_iota(jnp.int32, (T1_BLK, NUM_CAT), 1)
            == ids[:, 0:1]).astype(jnp.float32)
    oh_i = (lax.broadcasted_iota(jnp.int32, (T1_BLK, NUM_INGR), 1)
            == ids[:, 1:2]).astype(jnp.float32)
    oh_s = (lax.broadcasted_iota(jnp.int32, (T1_BLK, NUM_SHOP), 1)
            == ids[:, 2:3]).astype(jnp.float32)
    e_c = jnp.dot(oh_c, ce_ref[...], preferred_element_type=jnp.float32)
    e_i = jnp.dot(oh_i, ie_ref[...], preferred_element_type=jnp.float32)
    e_s = jnp.dot(oh_s, se_ref[...], preferred_element_type=jnp.float32)

    fused_in = jnp.concatenate([e_c, e_i, e_s], axis=1)
    fused = jnp.dot(fused_in, wf_ref[...],
                    preferred_element_type=jnp.float32,
                    precision=lax.Precision.HIGHEST) + bf_ref[...]
    fused = jnp.maximum(fused, 0.0)
    h0 = jnp.concatenate([x_ref[...], fused], axis=1)
    val1 = jnp.dot(h0, w1_ref[...],
                   preferred_element_type=jnp.float32,
                   precision=lax.Precision.HIGHEST)
    out_ref[0, :, :] = val1[:, :D]
    out_ref[1, :, :] = val1[:, D:]


def _ts_body(v_ref, d_ref, out_ref):
    deg = jnp.sum(d_ref[...], axis=1) + 1.0
    dinv = lax.rsqrt(deg)
    out_ref[...] = v_ref[...] * dinv[None, :, None]


def _t2_body(g_ref, d_ref, b1_ref, bn_g_ref, bn_b_ref, w2_ref, out_ref):
    deg = jnp.sum(d_ref[...], axis=1) + 1.0
    dinv = lax.rsqrt(deg)

    h1 = g_ref[0] * dinv[:, None] + b1_ref[0]
    mean = jnp.mean(h1, axis=0)
    var = jnp.mean((h1 - mean) ** 2, axis=0)
    hbn = (h1 - mean) * lax.rsqrt(var + 1e-5) * bn_g_ref[0] + bn_b_ref[0]
    hbn = jnp.maximum(hbn, 0.0)
    xw2 = jnp.dot(hbn, w2_ref[...],
                  preferred_element_type=jnp.float32,
                  precision=lax.Precision.HIGHEST) * dinv[:, None]

    @pl.when(pl.program_id(0) == 0)
    def _():
        out_ref[...] = xw2

    @pl.when(pl.program_id(0) != 0)
    def _():
        out_ref[...] += xw2


def _t3_body(acc_ref, val2_ref, d_ref, b2_ref, out_ref):
    deg = jnp.sum(d_ref[...], axis=1) + 1.0
    dinv = lax.rsqrt(deg)
    a = acc_ref[...]
    out_ref[...] = (a[0] + a[1] - val2_ref[...]) * dinv[:, None] + b2_ref[...]


def kernel(x, edge_index, cat_id, ingr_id, shop_id, cat_emb, ingr_emb,
           shop_emb, Wf, bf, W1, b1, gamma, beta, W2, b2):
    src = edge_index[0].astype(jnp.int32)
    dst = edge_index[1].astype(jnp.int32)
    ids = jnp.stack([cat_id.astype(jnp.int32), ingr_id.astype(jnp.int32),
                     shop_id.astype(jnp.int32)], axis=1)

    dacc = _make_sc_deg()(dst)
    deg2 = dacc[:, :, 0].T

    vals1_raw = pl.pallas_call(
        _t1_body,
        grid=(T1_STEPS,),
        in_specs=[
            pl.BlockSpec((T1_BLK, IN_DIM), lambda i: (i, 0)),
            pl.BlockSpec((T1_BLK, 3), lambda i: (i, 0)),
            pl.BlockSpec((NUM_CAT, EMB), lambda i: (0, 0)),
            pl.BlockSpec((NUM_INGR, EMB), lambda i: (0, 0)),
            pl.BlockSpec((NUM_SHOP, EMB), lambda i: (0, 0)),
            pl.BlockSpec((EMB * 3, FUSE), lambda i: (0, 0)),
            pl.BlockSpec((1, FUSE), lambda i: (0, 0)),
            pl.BlockSpec((D1, HIDDEN), lambda i: (0, 0)),
        ],
        out_specs=pl.BlockSpec((NC, T1_BLK, D), lambda i: (0, i, 0)),
        out_shape=jax.ShapeDtypeStruct((NC, N_NODES, D), jnp.float32),
    )(x, ids, cat_emb, ingr_emb, shop_emb, Wf, bf.reshape(1, FUSE), W1)

    vals1 = pl.pallas_call(
        _ts_body,
        grid=(T1_STEPS,),
        in_specs=[
            pl.BlockSpec((NC, T1_BLK, D), lambda i: (0, i, 0)),
            pl.BlockSpec((T1_BLK, NC), lambda i: (i, 0)),
        ],
        out_specs=pl.BlockSpec((NC, T1_BLK, D), lambda i: (0, i, 0)),
        out_shape=jax.ShapeDtypeStruct((NC, N_NODES, D), jnp.float32),
    )(vals1_raw, deg2)

    gs = _make_sc_agg_cols()(vals1, src, dst)
    val2 = pl.pallas_call(
        _t2_body,
        grid=(2,),
        in_specs=[
            pl.BlockSpec((1, N_NODES, D), lambda j: (j, 0, 0)),
            pl.BlockSpec((N_NODES, NC), lambda j: (0, 0)),
            pl.BlockSpec((1, 1, D), lambda j: (j, 0, 0)),
            pl.BlockSpec((1, 1, D), lambda j: (j, 0, 0)),
            pl.BlockSpec((1, 1, D), lambda j: (j, 0, 0)),
            pl.BlockSpec((D, OUT_DIM), lambda j: (j, 0)),
        ],
        out_specs=pl.BlockSpec((N_NODES, OUT_DIM), lambda j: (0, 0)),
        out_shape=jax.ShapeDtypeStruct((N_NODES, OUT_DIM), jnp.float32),
    )(gs, deg2, b1.reshape(2, 1, D), gamma.reshape(2, 1, D),
      beta.reshape(2, 1, D), W2)

    acc2 = _make_sc_agg()(val2, src, dst)

    out = pl.pallas_call(
        _t3_body,
        out_shape=jax.ShapeDtypeStruct((N_NODES, OUT_DIM), jnp.float32),
    )(acc2, val2, deg2, b2.reshape(1, OUT_DIM))
    return out
